# baseline (speedup 1.0000x reference)
import functools

import jax
import jax.numpy as jnp
from jax.experimental import pallas as pl
from jax.experimental.pallas import tpu as pltpu


def _conv_relu(x_ref, w_ref, *, K, L, R):
    pad = K // 2
    cin = x_ref.shape[1]
    z = jnp.zeros((cin, pad), jnp.bfloat16)
    cols = []
    for r in range(R):
        xp = jnp.concatenate([z, x_ref[r].astype(jnp.bfloat16), z], axis=1)
        cols.append(jnp.concatenate(
            [xp[:, k:k + L] for k in range(K)], axis=0))
    im2col = jnp.concatenate(cols, axis=1)
    acc = jax.lax.dot_general(
        w_ref[...], im2col,
        dimension_numbers=(((1,), (0,)), ((), ())),
        preferred_element_type=jnp.float32)
    return jnp.maximum(acc, 0.0)


def _stats_kernel(x_ref, w_ref, stats_ref, *, K, L, R):
    acc = _conv_relu(x_ref, w_ref, K=K, L=L, R=R)
    s = jnp.sum(acc, axis=1)
    s2 = jnp.sum(acc * acc, axis=1)
    stats_ref[0] = jnp.stack([s, s2], axis=0)


def _apply_kernel(x_ref, w_ref, stats_ref, g_ref, b_ref, o_ref,
                  *, K, L, R, count, eps):
    acc = _conv_relu(x_ref, w_ref, K=K, L=L, R=R)
    totals = jnp.sum(stats_ref[...], axis=0)
    mean = totals[0] / count
    var = totals[1] / count - mean * mean
    inv = jax.lax.rsqrt(var + eps)
    scale = (g_ref[0] * inv)[:, None]
    shift = (b_ref[0] - mean * g_ref[0] * inv)[:, None]
    out = acc * scale + shift
    for r in range(R):
        o_ref[r] = out[:, r * L:(r + 1) * L].astype(o_ref.dtype)


def _pick_rows(b):
    for r in (8, 4, 2):
        if b % r == 0:
            return r
    return 1


def kernel(x, weight, gamma, beta, *, eps=1e-5):
    B, Cin, L = x.shape
    Cout, _, K = weight.shape
    R = _pick_rows(B)
    nG = B // R

    w = jnp.transpose(weight, (0, 2, 1)).reshape(Cout, K * Cin).astype(jnp.bfloat16)

    stats = pl.pallas_call(
        functools.partial(_stats_kernel, K=K, L=L, R=R),
        out_shape=jax.ShapeDtypeStruct((nG, 2, Cout), jnp.float32),
        grid=(nG,),
        in_specs=[
            pl.BlockSpec((R, Cin, L), lambda g: (g, 0, 0)),
            pl.BlockSpec((Cout, K * Cin), lambda g: (0, 0)),
        ],
        out_specs=pl.BlockSpec((1, 2, Cout), lambda g: (g, 0, 0)),
        compiler_params=pltpu.CompilerParams(
            dimension_semantics=("parallel",),
            vmem_limit_bytes=100 * 1024 * 1024),
    )(x, w)

    out = pl.pallas_call(
        functools.partial(_apply_kernel, K=K, L=L, R=R,
                          count=float(B * L), eps=eps),
        out_shape=jax.ShapeDtypeStruct((B, Cout, L), x.dtype),
        grid=(nG,),
        in_specs=[
            pl.BlockSpec((R, Cin, L), lambda g: (g, 0, 0)),
            pl.BlockSpec((Cout, K * Cin), lambda g: (0, 0)),
            pl.BlockSpec((nG, 2, Cout), lambda g: (0, 0, 0)),
            pl.BlockSpec((1, Cout), lambda g: (0, 0)),
            pl.BlockSpec((1, Cout), lambda g: (0, 0)),
        ],
        out_specs=pl.BlockSpec((R, Cout, L), lambda g: (g, 0, 0)),
        compiler_params=pltpu.CompilerParams(
            dimension_semantics=("parallel",),
            vmem_limit_bytes=100 * 1024 * 1024),
    )(x, w, stats, gamma.reshape(1, Cout), beta.reshape(1, Cout))
    return out


# speedup vs baseline: 1.1872x; 1.1872x over previous
import functools

import jax
import jax.numpy as jnp
from jax.experimental import pallas as pl
from jax.experimental.pallas import tpu as pltpu


def _conv_relu_stats_kernel(x_ref, w_ref, y_ref, stats_ref, *, K, L, R):
    pad = K // 2
    cin = x_ref.shape[1]
    z = jnp.zeros((cin, pad), jnp.bfloat16)

    H = 2 if R % 2 == 0 else 1
    RH = R // H
    s_parts, s2_parts = [], []
    for h in range(H):
        cols = []
        for r in range(h * RH, (h + 1) * RH):
            xp = jnp.concatenate([z, x_ref[r].astype(jnp.bfloat16), z], axis=1)
            cols.append(jnp.concatenate(
                [xp[:, k:k + L] for k in range(K)], axis=0))
        im2col = jnp.concatenate(cols, axis=1)

        acc = jax.lax.dot_general(
            w_ref[...], im2col,
            dimension_numbers=(((1,), (0,)), ((), ())),
            preferred_element_type=jnp.float32)
        acc = jnp.maximum(acc, 0.0)

        for i in range(RH):
            y_ref[h * RH + i] = acc[:, i * L:(i + 1) * L].astype(y_ref.dtype)
        s_parts.append(jnp.sum(acc, axis=1))
        s2_parts.append(jnp.sum(acc * acc, axis=1))

    stats_ref[0] = jnp.stack([sum(s_parts), sum(s2_parts)], axis=0)


def _bn_apply_kernel(y_ref, stats_ref, g_ref, b_ref, o_ref, *, count, eps):
    totals = jnp.sum(stats_ref[...], axis=0)
    mean = totals[0] / count
    var = totals[1] / count - mean * mean
    inv = jax.lax.rsqrt(var + eps)
    scale = (g_ref[0] * inv)[None, :, None]
    shift = (b_ref[0] - mean * g_ref[0] * inv)[None, :, None]
    y = y_ref[...].astype(jnp.float32)
    o_ref[...] = (y * scale + shift).astype(o_ref.dtype)


def _pick_rows(b):
    for r in (16, 8, 4, 2):
        if b % r == 0:
            return r
    return 1


def kernel(x, weight, gamma, beta, *, eps=1e-5):
    B, Cin, L = x.shape
    Cout, _, K = weight.shape
    R = _pick_rows(B)
    nG = B // R

    w = jnp.transpose(weight, (0, 2, 1)).reshape(Cout, K * Cin).astype(jnp.bfloat16)

    conv = functools.partial(_conv_relu_stats_kernel, K=K, L=L, R=R)
    y, stats = pl.pallas_call(
        conv,
        out_shape=(
            jax.ShapeDtypeStruct((B, Cout, L), jnp.bfloat16),
            jax.ShapeDtypeStruct((nG, 2, Cout), jnp.float32),
        ),
        grid=(nG,),
        in_specs=[
            pl.BlockSpec((R, Cin, L), lambda g: (g, 0, 0)),
            pl.BlockSpec((Cout, K * Cin), lambda g: (0, 0)),
        ],
        out_specs=(
            pl.BlockSpec((R, Cout, L), lambda g: (g, 0, 0)),
            pl.BlockSpec((1, 2, Cout), lambda g: (g, 0, 0)),
        ),
        compiler_params=pltpu.CompilerParams(
            dimension_semantics=("parallel",),
            vmem_limit_bytes=100 * 1024 * 1024),
    )(x, w)

    bn = functools.partial(_bn_apply_kernel, count=float(B * L), eps=eps)
    out = pl.pallas_call(
        bn,
        out_shape=jax.ShapeDtypeStruct((B, Cout, L), x.dtype),
        grid=(nG,),
        in_specs=[
            pl.BlockSpec((R, Cout, L), lambda g: (g, 0, 0)),
            pl.BlockSpec((nG, 2, Cout), lambda g: (0, 0, 0)),
            pl.BlockSpec((1, Cout), lambda g: (0, 0)),
            pl.BlockSpec((1, Cout), lambda g: (0, 0)),
        ],
        out_specs=pl.BlockSpec((R, Cout, L), lambda g: (g, 0, 0)),
        compiler_params=pltpu.CompilerParams(
            dimension_semantics=("parallel",),
            vmem_limit_bytes=100 * 1024 * 1024),
    )(y, stats, gamma.reshape(1, Cout), beta.reshape(1, Cout))
    return out
